# baseline (device time: 46765 ns/iter reference)
import jax
import jax.numpy as jnp
from jax import lax
from jax.experimental import pallas as pl
from jax.experimental.pallas import tpu as pltpu

N_DEV = 4


def kernel(x, w_mat):
    m, k_per = x.shape
    _, n = w_mat.shape
    m_per = m // N_DEV

    def body(x_ref, w_ref, out_ref, send_buf, recv_buf, send_sems, recv_sems):
        my = lax.axis_index("i")
        left = lax.rem(my + N_DEV - 1, N_DEV)
        right = lax.rem(my + 1, N_DEV)

        barrier = pltpu.get_barrier_semaphore()
        for nbr in (left, right):
            pl.semaphore_signal(
                barrier, inc=1,
                device_id=(nbr,), device_id_type=pl.DeviceIdType.MESH,
            )
        pl.semaphore_wait(barrier, 2)

        def partial_chunk(c):
            xs = x_ref[pl.ds(c * m_per, m_per), :]
            return jnp.dot(xs, w_ref[:, :], preferred_element_type=jnp.float32)

        for h in range(N_DEV - 1):
            c = lax.rem(my + 2 * N_DEV - 1 - h, N_DEV)
            part = partial_chunk(c)
            if h == 0:
                send_buf[h, :, :] = part
            else:
                send_buf[h, :, :] = part + recv_buf[h - 1, :, :]
            rdma = pltpu.make_async_remote_copy(
                src_ref=send_buf.at[h],
                dst_ref=recv_buf.at[h],
                send_sem=send_sems.at[h],
                recv_sem=recv_sems.at[h],
                device_id=(right,),
                device_id_type=pl.DeviceIdType.MESH,
            )
            rdma.start()
            rdma.wait()

        part = partial_chunk(my)
        y = part + recv_buf[N_DEV - 2, :, :]
        out_ref[:, :] = y * (1.0 / (1.0 + jnp.exp(-y)))

    return pl.pallas_call(
        body,
        out_shape=jax.ShapeDtypeStruct((m_per, n), jnp.float32),
        in_specs=[
            pl.BlockSpec(memory_space=pltpu.VMEM),
            pl.BlockSpec(memory_space=pltpu.VMEM),
        ],
        out_specs=pl.BlockSpec(memory_space=pltpu.VMEM),
        scratch_shapes=[
            pltpu.VMEM((N_DEV - 1, m_per, n), jnp.float32),
            pltpu.VMEM((N_DEV - 1, m_per, n), jnp.float32),
            pltpu.SemaphoreType.DMA((N_DEV - 1,)),
            pltpu.SemaphoreType.DMA((N_DEV - 1,)),
        ],
        compiler_params=pltpu.CompilerParams(collective_id=0),
    )(x, w_mat)


# device time: 30177 ns/iter; 1.5497x vs baseline; 1.5497x over previous
import jax
import jax.numpy as jnp
from jax import lax
from jax.experimental import pallas as pl
from jax.experimental.pallas import tpu as pltpu

N_DEV = 4


def kernel(x, w_mat):
    m, k_per = x.shape
    _, n = w_mat.shape
    m_per = m // N_DEV
    n_half = n // 2

    def body(x_ref, w_ref, out_ref,
             send_a, recv_a, send_b, recv_b,
             send_a_sems, recv_a_sems, send_b_sems, recv_b_sems):
        my = lax.axis_index("i")
        left = lax.rem(my + N_DEV - 1, N_DEV)
        right = lax.rem(my + 1, N_DEV)

        barrier = pltpu.get_barrier_semaphore()
        for nbr in (left, right):
            pl.semaphore_signal(
                barrier, inc=1,
                device_id=(nbr,), device_id_type=pl.DeviceIdType.MESH,
            )
        pl.semaphore_wait(barrier, 2)

        def part_a(c):
            xs = x_ref[pl.ds(c * m_per, m_per), :]
            return jnp.dot(xs, w_ref[:, :n_half],
                           preferred_element_type=jnp.float32)

        def part_b(c):
            xs = x_ref[pl.ds(c * m_per, m_per), :]
            return jnp.dot(xs, w_ref[:, n_half:],
                           preferred_element_type=jnp.float32)

        def cw_chunk(h):
            return lax.rem(my + 2 * N_DEV - 1 - h, N_DEV)

        def ccw_chunk(h):
            return lax.rem(my + 1 + h, N_DEV)

        def make_rdmas(h):
            a = pltpu.make_async_remote_copy(
                src_ref=send_a.at[h], dst_ref=recv_a.at[h],
                send_sem=send_a_sems.at[h], recv_sem=recv_a_sems.at[h],
                device_id=(right,), device_id_type=pl.DeviceIdType.MESH,
            )
            b = pltpu.make_async_remote_copy(
                src_ref=send_b.at[h], dst_ref=recv_b.at[h],
                send_sem=send_b_sems.at[h], recv_sem=recv_b_sems.at[h],
                device_id=(left,), device_id_type=pl.DeviceIdType.MESH,
            )
            return a, b

        send_a[0, :, :] = part_a(cw_chunk(0))
        send_b[0, :, :] = part_b(ccw_chunk(0))
        rd_a0, rd_b0 = make_rdmas(0)
        rd_a0.start()
        rd_b0.start()

        pa1 = part_a(cw_chunk(1))
        pb1 = part_b(ccw_chunk(1))

        rd_a0.wait()
        rd_b0.wait()
        send_a[1, :, :] = pa1 + recv_a[0, :, :]
        send_b[1, :, :] = pb1 + recv_b[0, :, :]
        rd_a1, rd_b1 = make_rdmas(1)
        rd_a1.start()
        rd_b1.start()

        pa2 = part_a(cw_chunk(2))
        pb2 = part_b(ccw_chunk(2))
        own_a = part_a(my)
        own_b = part_b(my)

        rd_a1.wait()
        rd_b1.wait()
        send_a[2, :, :] = pa2 + recv_a[1, :, :]
        send_b[2, :, :] = pb2 + recv_b[1, :, :]
        rd_a2, rd_b2 = make_rdmas(2)
        rd_a2.start()
        rd_b2.start()

        rd_a2.wait()
        rd_b2.wait()
        ya = own_a + recv_a[2, :, :]
        yb = own_b + recv_b[2, :, :]
        out_ref[:, :n_half] = ya * (1.0 / (1.0 + jnp.exp(-ya)))
        out_ref[:, n_half:] = yb * (1.0 / (1.0 + jnp.exp(-yb)))

    return pl.pallas_call(
        body,
        out_shape=jax.ShapeDtypeStruct((m_per, n), jnp.float32),
        in_specs=[
            pl.BlockSpec(memory_space=pltpu.VMEM),
            pl.BlockSpec(memory_space=pltpu.VMEM),
        ],
        out_specs=pl.BlockSpec(memory_space=pltpu.VMEM),
        scratch_shapes=[
            pltpu.VMEM((N_DEV - 1, m_per, n_half), jnp.float32),
            pltpu.VMEM((N_DEV - 1, m_per, n_half), jnp.float32),
            pltpu.VMEM((N_DEV - 1, m_per, n_half), jnp.float32),
            pltpu.VMEM((N_DEV - 1, m_per, n_half), jnp.float32),
            pltpu.SemaphoreType.DMA((N_DEV - 1,)),
            pltpu.SemaphoreType.DMA((N_DEV - 1,)),
            pltpu.SemaphoreType.DMA((N_DEV - 1,)),
            pltpu.SemaphoreType.DMA((N_DEV - 1,)),
        ],
        compiler_params=pltpu.CompilerParams(collective_id=0),
    )(x, w_mat)


# device time: 25933 ns/iter; 1.8033x vs baseline; 1.1637x over previous
import jax
import jax.numpy as jnp
from jax import lax
from jax.experimental import pallas as pl
from jax.experimental.pallas import tpu as pltpu

N_DEV = 4
N_SUB = 2


def kernel(x, w_mat):
    m, k_per = x.shape
    _, n = w_mat.shape
    m_per = m // N_DEV
    n_half = n // 2
    m_sub = m_per // N_SUB

    def body(x_ref, w_ref, out_ref,
             send_a, recv_a, send_b, recv_b,
             send_a_sems, recv_a_sems, send_b_sems, recv_b_sems):
        my = lax.axis_index("i")
        left = lax.rem(my + N_DEV - 1, N_DEV)
        right = lax.rem(my + 1, N_DEV)

        barrier = pltpu.get_barrier_semaphore()
        for nbr in (left, right):
            pl.semaphore_signal(
                barrier, inc=1,
                device_id=(nbr,), device_id_type=pl.DeviceIdType.MESH,
            )
        pl.semaphore_wait(barrier, 2)

        def part_a(c):
            xs = x_ref[pl.ds(c * m_per, m_per), :]
            return jnp.dot(xs, w_ref[:, :n_half],
                           preferred_element_type=jnp.float32)

        def part_b(c):
            xs = x_ref[pl.ds(c * m_per, m_per), :]
            return jnp.dot(xs, w_ref[:, n_half:],
                           preferred_element_type=jnp.float32)

        def cw_chunk(h):
            return lax.rem(my + 2 * N_DEV - 1 - h, N_DEV)

        def ccw_chunk(h):
            return lax.rem(my + 1 + h, N_DEV)

        def rdma_a(h, j):
            return pltpu.make_async_remote_copy(
                src_ref=send_a.at[h, j], dst_ref=recv_a.at[h, j],
                send_sem=send_a_sems.at[h, j], recv_sem=recv_a_sems.at[h, j],
                device_id=(right,), device_id_type=pl.DeviceIdType.MESH,
            )

        def rdma_b(h, j):
            return pltpu.make_async_remote_copy(
                src_ref=send_b.at[h, j], dst_ref=recv_b.at[h, j],
                send_sem=send_b_sems.at[h, j], recv_sem=recv_b_sems.at[h, j],
                device_id=(left,), device_id_type=pl.DeviceIdType.MESH,
            )

        pa0 = part_a(cw_chunk(0))
        for j in range(N_SUB):
            send_a[0, j, :, :] = pa0[j * m_sub:(j + 1) * m_sub, :]
            rdma_a(0, j).start()
        pb0 = part_b(ccw_chunk(0))
        for j in range(N_SUB):
            send_b[0, j, :, :] = pb0[j * m_sub:(j + 1) * m_sub, :]
            rdma_b(0, j).start()

        pa1 = part_a(cw_chunk(1))
        pb1 = part_b(ccw_chunk(1))

        for j in range(N_SUB):
            ra = rdma_a(0, j)
            ra.wait()
            send_a[1, j, :, :] = (
                pa1[j * m_sub:(j + 1) * m_sub, :] + recv_a[0, j, :, :]
            )
            rdma_a(1, j).start()
            rb = rdma_b(0, j)
            rb.wait()
            send_b[1, j, :, :] = (
                pb1[j * m_sub:(j + 1) * m_sub, :] + recv_b[0, j, :, :]
            )
            rdma_b(1, j).start()

        pa2 = part_a(cw_chunk(2))
        pb2 = part_b(ccw_chunk(2))

        for j in range(N_SUB):
            ra = rdma_a(1, j)
            ra.wait()
            send_a[2, j, :, :] = (
                pa2[j * m_sub:(j + 1) * m_sub, :] + recv_a[1, j, :, :]
            )
            rdma_a(2, j).start()
            rb = rdma_b(1, j)
            rb.wait()
            send_b[2, j, :, :] = (
                pb2[j * m_sub:(j + 1) * m_sub, :] + recv_b[1, j, :, :]
            )
            rdma_b(2, j).start()

        own_a = part_a(my)
        own_b = part_b(my)

        for j in range(N_SUB):
            ra = rdma_a(2, j)
            ra.wait()
            ya = own_a[j * m_sub:(j + 1) * m_sub, :] + recv_a[2, j, :, :]
            out_ref[j * m_sub:(j + 1) * m_sub, :n_half] = (
                ya * (1.0 / (1.0 + jnp.exp(-ya)))
            )
            rb = rdma_b(2, j)
            rb.wait()
            yb = own_b[j * m_sub:(j + 1) * m_sub, :] + recv_b[2, j, :, :]
            out_ref[j * m_sub:(j + 1) * m_sub, n_half:] = (
                yb * (1.0 / (1.0 + jnp.exp(-yb)))
            )

    return pl.pallas_call(
        body,
        out_shape=jax.ShapeDtypeStruct((m_per, n), jnp.float32),
        in_specs=[
            pl.BlockSpec(memory_space=pltpu.VMEM),
            pl.BlockSpec(memory_space=pltpu.VMEM),
        ],
        out_specs=pl.BlockSpec(memory_space=pltpu.VMEM),
        scratch_shapes=[
            pltpu.VMEM((N_DEV - 1, N_SUB, m_sub, n_half), jnp.float32),
            pltpu.VMEM((N_DEV - 1, N_SUB, m_sub, n_half), jnp.float32),
            pltpu.VMEM((N_DEV - 1, N_SUB, m_sub, n_half), jnp.float32),
            pltpu.VMEM((N_DEV - 1, N_SUB, m_sub, n_half), jnp.float32),
            pltpu.SemaphoreType.DMA((N_DEV - 1, N_SUB)),
            pltpu.SemaphoreType.DMA((N_DEV - 1, N_SUB)),
            pltpu.SemaphoreType.DMA((N_DEV - 1, N_SUB)),
            pltpu.SemaphoreType.DMA((N_DEV - 1, N_SUB)),
        ],
        compiler_params=pltpu.CompilerParams(collective_id=0),
    )(x, w_mat)


# device time: 17646 ns/iter; 2.6502x vs baseline; 1.4696x over previous
import jax
import jax.numpy as jnp
from jax import lax
from jax.experimental import pallas as pl
from jax.experimental.pallas import tpu as pltpu

N_DEV = 4
N_SUB = 4


def kernel(x, w_mat):
    m, k_per = x.shape
    _, n = w_mat.shape
    m_per = m // N_DEV
    n_half = n // 2
    m_sub = m_per // N_SUB

    def body(x_ref, w_ref, out_ref,
             send_a, recv_a, send_b, recv_b,
             send_a_sems, recv_a_sems, send_b_sems, recv_b_sems):
        my = lax.axis_index("i")
        left = lax.rem(my + N_DEV - 1, N_DEV)
        right = lax.rem(my + 1, N_DEV)

        barrier = pltpu.get_barrier_semaphore()
        for nbr in (left, right):
            pl.semaphore_signal(
                barrier, inc=1,
                device_id=(nbr,), device_id_type=pl.DeviceIdType.MESH,
            )
        pl.semaphore_wait(barrier, 2)

        def part_a(c):
            xs = x_ref[pl.ds(c * m_per, m_per), :]
            return jnp.dot(xs, w_ref[:, :n_half],
                           preferred_element_type=jnp.float32)

        def part_b(c):
            xs = x_ref[pl.ds(c * m_per, m_per), :]
            return jnp.dot(xs, w_ref[:, n_half:],
                           preferred_element_type=jnp.float32)

        def cw_chunk(h):
            return lax.rem(my + 2 * N_DEV - 1 - h, N_DEV)

        def ccw_chunk(h):
            return lax.rem(my + 1 + h, N_DEV)

        def rdma_a(h, j):
            return pltpu.make_async_remote_copy(
                src_ref=send_a.at[h, j], dst_ref=recv_a.at[h, j],
                send_sem=send_a_sems.at[h, j], recv_sem=recv_a_sems.at[h, j],
                device_id=(right,), device_id_type=pl.DeviceIdType.MESH,
            )

        def rdma_b(h, j):
            return pltpu.make_async_remote_copy(
                src_ref=send_b.at[h, j], dst_ref=recv_b.at[h, j],
                send_sem=send_b_sems.at[h, j], recv_sem=recv_b_sems.at[h, j],
                device_id=(left,), device_id_type=pl.DeviceIdType.MESH,
            )

        pa0 = part_a(cw_chunk(0))
        for j in range(N_SUB):
            send_a[0, j, :, :] = (
                pa0[j * m_sub:(j + 1) * m_sub, :].astype(jnp.bfloat16)
            )
            rdma_a(0, j).start()
        pb0 = part_b(ccw_chunk(0))
        for j in range(N_SUB):
            send_b[0, j, :, :] = (
                pb0[j * m_sub:(j + 1) * m_sub, :].astype(jnp.bfloat16)
            )
            rdma_b(0, j).start()

        pa1 = part_a(cw_chunk(1))
        pb1 = part_b(ccw_chunk(1))

        for j in range(N_SUB):
            ra = rdma_a(0, j)
            ra.wait()
            send_a[1, j, :, :] = (
                pa1[j * m_sub:(j + 1) * m_sub, :] + recv_a[0, j, :, :]
            ).astype(jnp.bfloat16)
            rdma_a(1, j).start()
            rb = rdma_b(0, j)
            rb.wait()
            send_b[1, j, :, :] = (
                pb1[j * m_sub:(j + 1) * m_sub, :] + recv_b[0, j, :, :]
            ).astype(jnp.bfloat16)
            rdma_b(1, j).start()

        pa2 = part_a(cw_chunk(2))
        pb2 = part_b(ccw_chunk(2))

        for j in range(N_SUB):
            ra = rdma_a(1, j)
            ra.wait()
            send_a[2, j, :, :] = (
                pa2[j * m_sub:(j + 1) * m_sub, :] + recv_a[1, j, :, :]
            ).astype(jnp.bfloat16)
            rdma_a(2, j).start()
            rb = rdma_b(1, j)
            rb.wait()
            send_b[2, j, :, :] = (
                pb2[j * m_sub:(j + 1) * m_sub, :] + recv_b[1, j, :, :]
            ).astype(jnp.bfloat16)
            rdma_b(2, j).start()

        own_a = part_a(my)
        own_b = part_b(my)

        for j in range(N_SUB):
            ra = rdma_a(2, j)
            ra.wait()
            ya = own_a[j * m_sub:(j + 1) * m_sub, :] + recv_a[2, j, :, :]
            out_ref[j * m_sub:(j + 1) * m_sub, :n_half] = (
                ya * (1.0 / (1.0 + jnp.exp(-ya)))
            )
            rb = rdma_b(2, j)
            rb.wait()
            yb = own_b[j * m_sub:(j + 1) * m_sub, :] + recv_b[2, j, :, :]
            out_ref[j * m_sub:(j + 1) * m_sub, n_half:] = (
                yb * (1.0 / (1.0 + jnp.exp(-yb)))
            )

    return pl.pallas_call(
        body,
        out_shape=jax.ShapeDtypeStruct((m_per, n), jnp.float32),
        in_specs=[
            pl.BlockSpec(memory_space=pltpu.VMEM),
            pl.BlockSpec(memory_space=pltpu.VMEM),
        ],
        out_specs=pl.BlockSpec(memory_space=pltpu.VMEM),
        scratch_shapes=[
            pltpu.VMEM((N_DEV - 1, N_SUB, m_sub, n_half), jnp.bfloat16),
            pltpu.VMEM((N_DEV - 1, N_SUB, m_sub, n_half), jnp.bfloat16),
            pltpu.VMEM((N_DEV - 1, N_SUB, m_sub, n_half), jnp.bfloat16),
            pltpu.VMEM((N_DEV - 1, N_SUB, m_sub, n_half), jnp.bfloat16),
            pltpu.SemaphoreType.DMA((N_DEV - 1, N_SUB)),
            pltpu.SemaphoreType.DMA((N_DEV - 1, N_SUB)),
            pltpu.SemaphoreType.DMA((N_DEV - 1, N_SUB)),
            pltpu.SemaphoreType.DMA((N_DEV - 1, N_SUB)),
        ],
        compiler_params=pltpu.CompilerParams(collective_id=0),
    )(x, w_mat)
